# baseline (speedup 1.0000x reference)
import functools

import jax
import jax.numpy as jnp
from jax import lax
from jax.experimental import pallas as pl
from jax.experimental.pallas import tpu as pltpu
from jax.experimental.pallas import tpu_sc as plsc

EMB = 64
BATCH = 4096
OUT = 128
NC, NS = 2, 16
NW = NC * NS
BPW = BATCH // NW

_sc_mesh = plsc.VectorSubcoreMesh(core_axis_name="c", subcore_axis_name="s")


@functools.partial(
    pl.kernel,
    out_type=tuple(
        jax.ShapeDtypeStruct((BATCH, EMB), jnp.float32) for _ in range(4)
    ),
    mesh=_sc_mesh,
    compiler_params=pltpu.CompilerParams(use_tc_tiling_on_sc=False),
    scratch_types=[
        pltpu.VMEM((BPW,), jnp.int32),
        pltpu.VMEM((BPW, EMB), jnp.float32),
        pltpu.SemaphoreType.DMA,
    ],
)
def _sc_gather4(xa, xb, xc, xd, Ta, Tb, Tc, Td, ea, eb, ec, ed,
                idx_v, rows_v, sem):
    wid = lax.axis_index("s") * NC + lax.axis_index("c")
    base = wid * BPW
    for x, T, e in ((xa, Ta, ea), (xb, Tb, eb), (xc, Tc, ec), (xd, Td, ed)):
        pltpu.sync_copy(x.at[pl.ds(base, BPW)], idx_v)
        pltpu.async_copy(T.at[idx_v], rows_v, sem).wait()
        pltpu.sync_copy(rows_v, e.at[pl.ds(base, BPW)])


_BM = 512


def _tc_body(ea, eb, ec, ed, w_ref, b_ref, out_ref):
    acc = jnp.dot(ea[...], w_ref[0 * EMB:1 * EMB, :],
                  preferred_element_type=jnp.float32)
    acc = acc + jnp.dot(eb[...], w_ref[1 * EMB:2 * EMB, :],
                        preferred_element_type=jnp.float32)
    acc = acc + jnp.dot(ec[...], w_ref[2 * EMB:3 * EMB, :],
                        preferred_element_type=jnp.float32)
    acc = acc + jnp.dot(ed[...], w_ref[3 * EMB:4 * EMB, :],
                        preferred_element_type=jnp.float32)
    out_ref[...] = acc + b_ref[...]


def _tc_project(ea, eb, ec, ed, W, b2d):
    e_spec = pl.BlockSpec((_BM, EMB), lambda i: (i, 0))
    return pl.pallas_call(
        _tc_body,
        grid=(BATCH // _BM,),
        in_specs=[e_spec, e_spec, e_spec, e_spec,
                  pl.BlockSpec((4 * EMB, OUT), lambda i: (0, 0)),
                  pl.BlockSpec((1, OUT), lambda i: (0, 0))],
        out_specs=pl.BlockSpec((_BM, OUT), lambda i: (i, 0)),
        out_shape=jax.ShapeDtypeStruct((BATCH, OUT), jnp.float32),
    )(ea, eb, ec, ed, W, b2d)


def kernel(x_a, x_b, x_c, x_d, T_a, T_b, T_c, T_d, W, b):
    xs = [x.astype(jnp.int32) for x in (x_a, x_b, x_c, x_d)]
    ea, eb, ec, ed = _sc_gather4(*xs, T_a, T_b, T_c, T_d)
    return _tc_project(ea, eb, ec, ed, W, b.reshape(1, OUT))


# speedup vs baseline: 1.0179x; 1.0179x over previous
import functools

import jax
import jax.numpy as jnp
from jax import lax
from jax.experimental import pallas as pl
from jax.experimental.pallas import tpu as pltpu
from jax.experimental.pallas import tpu_sc as plsc

EMB = 64
BATCH = 4096
OUT = 128
VOCAB = 100000
NC, NS = 2, 16
NW = NC * NS
BPW = BATCH // NW
L = 16

_sc_mesh = plsc.VectorSubcoreMesh(core_axis_name="c", subcore_axis_name="s")


@functools.partial(
    pl.kernel,
    out_type=tuple(
        jax.ShapeDtypeStruct((BATCH, 2 * EMB), jnp.float32) for _ in range(4)
    ),
    mesh=_sc_mesh,
    scratch_types=[
        pltpu.VMEM((BPW,), jnp.int32),
        pltpu.VMEM((BPW,), jnp.int32),
        pltpu.VMEM((BPW, 2 * EMB), jnp.float32),
        pltpu.SemaphoreType.DMA,
    ],
)
def _sc_pairgather4(xa, xb, xc, xd, Ta, Tb, Tc, Td, ga, gb, gc, gd,
                    idx_v, j_v, rows_v, sem):
    wid = lax.axis_index("s") * NC + lax.axis_index("c")
    base = wid * BPW
    for x, T, g in ((xa, Ta, ga), (xb, Tb, gb), (xc, Tc, gc), (xd, Td, gd)):
        pltpu.sync_copy(x.at[pl.ds(base, BPW)], idx_v)
        for i in range(BPW // L):
            j_v[pl.ds(i * L, L)] = lax.shift_right_logical(
                idx_v[pl.ds(i * L, L)], 1
            )
        pltpu.async_copy(T.at[j_v], rows_v, sem).wait()
        pltpu.sync_copy(rows_v, g.at[pl.ds(base, BPW)])


_BM = 512


def _tc_body(ga, gb, gc, gd, xa, xb, xc, xd, w_ref, b_ref, out_ref):
    lane = lax.broadcasted_iota(jnp.int32, (_BM, 2 * EMB), 1)
    low = lane < EMB
    acc = b_ref[...] + jnp.zeros((_BM, OUT), jnp.float32)
    for g, x, t in ((ga, xa, 0), (gb, xb, 1), (gc, xc, 2), (gd, xd, 3)):
        par = (x[...] & 1) == 0
        mask = jnp.where(low == par, 1.0, 0.0)
        wt = w_ref[pl.ds(t * 2 * EMB, 2 * EMB), :]
        acc = acc + jnp.dot(g[...] * mask, wt,
                            preferred_element_type=jnp.float32)
    out_ref[...] = acc


def _tc_project(gs, xs2d, Wstack, b2d):
    g_spec = pl.BlockSpec((_BM, 2 * EMB), lambda i: (i, 0))
    x_spec = pl.BlockSpec((_BM, 1), lambda i: (i, 0))
    return pl.pallas_call(
        _tc_body,
        grid=(BATCH // _BM,),
        in_specs=[g_spec] * 4 + [x_spec] * 4 + [
            pl.BlockSpec((8 * EMB, OUT), lambda i: (0, 0)),
            pl.BlockSpec((1, OUT), lambda i: (0, 0))],
        out_specs=pl.BlockSpec((_BM, OUT), lambda i: (i, 0)),
        out_shape=jax.ShapeDtypeStruct((BATCH, OUT), jnp.float32),
    )(*gs, *xs2d, Wstack, b2d)


def kernel(x_a, x_b, x_c, x_d, T_a, T_b, T_c, T_d, W, b):
    xs = [x.astype(jnp.int32) for x in (x_a, x_b, x_c, x_d)]
    Tps = [T.reshape(VOCAB // 2, 2 * EMB) for T in (T_a, T_b, T_c, T_d)]
    gs = _sc_pairgather4(*xs, *Tps)
    Wstack = jnp.concatenate(
        [jnp.concatenate([W[t * EMB:(t + 1) * EMB]] * 2, axis=0)
         for t in range(4)], axis=0)
    xs2d = [x.reshape(BATCH, 1) for x in xs]
    return _tc_project(gs, xs2d, Wstack, b.reshape(1, OUT))


# speedup vs baseline: 1.8340x; 1.8017x over previous
import functools

import jax
import jax.numpy as jnp
from jax import lax
from jax.experimental import pallas as pl
from jax.experimental.pallas import tpu as pltpu
from jax.experimental.pallas import tpu_sc as plsc

EMB = 64
BATCH = 4096
OUT = 128
VOCAB = 100000
NC, NS = 2, 16
NW = NC * NS
BPW = BATCH // NW

_BV = 2048


def _proj_body(ta, tb, tc, td, w_ref, pa, pb, pc, pd):
    dn = (((0,), (0,)), ((), ()))
    for t, (tin, pout) in enumerate(((ta, pa), (tb, pb), (tc, pc), (td, pd))):
        pout[...] = lax.dot_general(
            tin[...], w_ref[pl.ds(t * EMB, EMB), :], dn,
            preferred_element_type=jnp.float32)


def _tc_project(Tts, W):
    t_spec = pl.BlockSpec((EMB, _BV), lambda i: (0, i))
    p_spec = pl.BlockSpec((_BV, OUT), lambda i: (i, 0))
    return pl.pallas_call(
        _proj_body,
        grid=((VOCAB + _BV - 1) // _BV,),
        in_specs=[t_spec] * 4 + [pl.BlockSpec((4 * EMB, OUT), lambda i: (0, 0))],
        out_specs=[p_spec] * 4,
        out_shape=[jax.ShapeDtypeStruct((VOCAB, OUT), jnp.float32)] * 4,
    )(*Tts, W)


_sc_mesh = plsc.VectorSubcoreMesh(core_axis_name="c", subcore_axis_name="s")


@functools.partial(
    pl.kernel,
    out_type=tuple(
        jax.ShapeDtypeStruct((BATCH, OUT), jnp.float32) for _ in range(4)
    ),
    mesh=_sc_mesh,
    scratch_types=[
        pltpu.VMEM((BPW,), jnp.int32),
        pltpu.VMEM((BPW, OUT), jnp.float32),
        pltpu.SemaphoreType.DMA,
    ],
)
def _sc_gather4(xa, xb, xc, xd, Pa, Pb, Pc, Pd, ga, gb, gc, gd,
                idx_v, rows_v, sem):
    wid = lax.axis_index("s") * NC + lax.axis_index("c")
    base = wid * BPW
    for x, P, g in ((xa, Pa, ga), (xb, Pb, gb), (xc, Pc, gc), (xd, Pd, gd)):
        pltpu.sync_copy(x.at[pl.ds(base, BPW)], idx_v)
        pltpu.async_copy(P.at[idx_v], rows_v, sem).wait()
        pltpu.sync_copy(rows_v, g.at[pl.ds(base, BPW)])


_BM = 1024


def _sum_body(ga, gb, gc, gd, b_ref, out_ref):
    out_ref[...] = (ga[...] + gb[...]) + (gc[...] + gd[...]) + b_ref[...]


def _tc_sum(gs, b2d):
    g_spec = pl.BlockSpec((_BM, OUT), lambda i: (i, 0))
    return pl.pallas_call(
        _sum_body,
        grid=(BATCH // _BM,),
        in_specs=[g_spec] * 4 + [pl.BlockSpec((1, OUT), lambda i: (0, 0))],
        out_specs=g_spec,
        out_shape=jax.ShapeDtypeStruct((BATCH, OUT), jnp.float32),
    )(*gs, b2d)


def kernel(x_a, x_b, x_c, x_d, T_a, T_b, T_c, T_d, W, b):
    xs = [x.astype(jnp.int32) for x in (x_a, x_b, x_c, x_d)]
    Ps = _tc_project([T.T for T in (T_a, T_b, T_c, T_d)], W)
    gs = _sc_gather4(*xs, *Ps)
    return _tc_sum(gs, b.reshape(1, OUT))
